# Initial kernel scaffold; baseline (speedup 1.0000x reference)
import jax
import jax.numpy as jnp
from jax.experimental import pallas as pl


def kernel(x, edge_index, batch, W_enc, b_enc, msg1_W, msg1_b, msg2_W, msg2_b, upd1_W, upd1_b, upd2_W, upd2_b, W_dec, b_dec):
    raise NotImplementedError("write your pallas kernel here")


# speedup vs baseline: 2.5004x; 2.5004x over previous
import functools

import jax
import jax.numpy as jnp
from jax import lax
from jax.experimental import pallas as pl
from jax.experimental.pallas import tpu as pltpu
from jax.experimental.pallas import tpu_sc as plsc

N = 10000
E = 320000
H = 128
NL = 4
NB = 10
EPS = 1e-5

NC = 2
NS = 16
NW = NC * NS
CH = 80
EPT = E // NW
E2 = E // NC
NACC = 10240
NPT = NACC // NS

_f32 = jnp.float32


def _enc_body(x_ref, we_ref, be_ref, w1a_ref, w1b_ref, b1_ref,
              h_ref, a_ref, b_ref):
    h = jnp.dot(x_ref[...], we_ref[...], preferred_element_type=_f32)
    h = h + be_ref[...]
    h_ref[...] = h
    a_ref[...] = jnp.dot(h, w1a_ref[...], preferred_element_type=_f32) + b1_ref[...]
    b_ref[...] = jnp.dot(h, w1b_ref[...], preferred_element_type=_f32)


_enc = pl.pallas_call(
    _enc_body,
    out_shape=[jax.ShapeDtypeStruct((N, H), _f32)] * 3,
)

BE = 2000


def _edge_body(ag_ref, bg_ref, w2_ref, b2_ref, m_ref):
    z = jnp.maximum(ag_ref[...] + bg_ref[...], 0.0)
    mm = jnp.dot(z, w2_ref[...], preferred_element_type=_f32)
    m_ref[...] = jnp.maximum(mm + b2_ref[...], 0.0)


_edge = pl.pallas_call(
    _edge_body,
    grid=(E // BE,),
    in_specs=[
        pl.BlockSpec((BE, H), lambda i: (i, 0)),
        pl.BlockSpec((BE, H), lambda i: (i, 0)),
        pl.BlockSpec((H, H), lambda i: (0, 0)),
        pl.BlockSpec((1, H), lambda i: (0, 0)),
    ],
    out_specs=pl.BlockSpec((BE, H), lambda i: (i, 0)),
    out_shape=jax.ShapeDtypeStruct((E, H), _f32),
)


def _norm(h_ref, p_ref, u1a_ref, u1b_ref, bu1_ref, u2_ref, bu2_ref,
          batc_ref, batr_ref):
    agg = p_ref[0, :N] + p_ref[1, :N]
    u = jnp.dot(h_ref[...], u1a_ref[...], preferred_element_type=_f32)
    u = u + jnp.dot(agg, u1b_ref[...], preferred_element_type=_f32)
    u = jnp.maximum(u + bu1_ref[...], 0.0)
    u = jnp.dot(u, u2_ref[...], preferred_element_type=_f32)
    u = jnp.maximum(u + bu2_ref[...], 0.0)
    oh = (batc_ref[...] == lax.broadcasted_iota(jnp.int32, (N, NB), 1)).astype(_f32)
    oht = (batr_ref[0:1] == lax.broadcasted_iota(jnp.int32, (NB, N), 0)).astype(_f32)
    cnt = jnp.dot(oht, jnp.ones((N, 1), _f32), preferred_element_type=_f32)
    cnt = jnp.maximum(cnt, 1.0)
    mean = jnp.dot(oht, u, preferred_element_type=_f32) / cnt
    uc = u - jnp.dot(oh, mean, preferred_element_type=_f32)
    var = jnp.dot(oht, uc * uc, preferred_element_type=_f32) / cnt
    return uc * lax.rsqrt(jnp.dot(oh, var, preferred_element_type=_f32) + EPS)


def _upd_body(h_ref, p_ref, u1a_ref, u1b_ref, bu1_ref, u2_ref, bu2_ref,
              batc_ref, batr_ref, wna_ref, wnb_ref, bn_ref,
              h2_ref, a_ref, b_ref):
    hn = _norm(h_ref, p_ref, u1a_ref, u1b_ref, bu1_ref, u2_ref, bu2_ref,
               batc_ref, batr_ref)
    h2_ref[...] = hn
    a_ref[...] = jnp.dot(hn, wna_ref[...], preferred_element_type=_f32) + bn_ref[...]
    b_ref[...] = jnp.dot(hn, wnb_ref[...], preferred_element_type=_f32)


_upd = pl.pallas_call(
    _upd_body,
    out_shape=[jax.ShapeDtypeStruct((N, H), _f32)] * 3,
)


def _fin_body(h_ref, p_ref, u1a_ref, u1b_ref, bu1_ref, u2_ref, bu2_ref,
              batc_ref, batr_ref, wd_ref, bd_ref, y_ref):
    hn = _norm(h_ref, p_ref, u1a_ref, u1b_ref, bu1_ref, u2_ref, bu2_ref,
               batc_ref, batr_ref)
    y_ref[...] = jnp.dot(hn, wd_ref[...], preferred_element_type=_f32) + bd_ref[...]


_fin = pl.pallas_call(
    _fin_body,
    out_shape=jax.ShapeDtypeStruct((N, H), _f32),
)


_vmesh = plsc.VectorSubcoreMesh(core_axis_name="c", subcore_axis_name="s")


@functools.partial(
    pl.kernel,
    mesh=_vmesh,
    out_type=[jax.ShapeDtypeStruct((E, H), _f32)] * 2,
    scratch_types=[
        pltpu.VMEM((CH,), jnp.int32),
        pltpu.VMEM((CH,), jnp.int32),
        pltpu.VMEM((CH, H), _f32),
        pltpu.VMEM((CH, H), _f32),
        pltpu.SemaphoreType.DMA,
        pltpu.SemaphoreType.DMA,
    ],
)
def _gather2(a_hbm, b_hbm, dst_hbm, src_hbm, ag_hbm, bg_hbm,
             idxd, idxs, bufa, bufb, sema, semb):
    wid = lax.axis_index("s") * NC + lax.axis_index("c")
    base = wid * EPT

    @pl.loop(0, EPT // CH)
    def _(k):
        off = base + k * CH
        pltpu.sync_copy(dst_hbm.at[pl.ds(off, CH)], idxd)
        pltpu.sync_copy(src_hbm.at[pl.ds(off, CH)], idxs)
        ca = pltpu.async_copy(a_hbm.at[idxd], bufa, sema)
        cb = pltpu.async_copy(b_hbm.at[idxs], bufb, semb)
        ca.wait()
        cb.wait()
        pltpu.sync_copy(bufa, ag_hbm.at[pl.ds(off, CH)])
        pltpu.sync_copy(bufb, bg_hbm.at[pl.ds(off, CH)])


@functools.partial(
    pl.kernel,
    mesh=_vmesh,
    out_type=jax.ShapeDtypeStruct((NC, NACC, H), _f32),
    scratch_types=[
        pltpu.VMEM((CH,), jnp.int32),
        pltpu.VMEM((CH, H), _f32),
        pltpu.VMEM_SHARED((NACC, H), _f32),
        pltpu.SemaphoreType.DMA,
    ],
)
def _scatter_add(m_hbm, dst_hbm, zeros_hbm, out_hbm, idx, buf, acc, sem):
    c = lax.axis_index("c")
    s = lax.axis_index("s")
    pltpu.sync_copy(zeros_hbm.at[pl.ds(s * NPT, NPT)], acc.at[pl.ds(s * NPT, NPT)])
    plsc.subcore_barrier()
    base = c * E2 + s * EPT

    @pl.loop(0, EPT // CH)
    def _(k):
        off = base + k * CH
        pltpu.sync_copy(dst_hbm.at[pl.ds(off, CH)], idx)
        pltpu.sync_copy(m_hbm.at[pl.ds(off, CH)], buf)
        pltpu.sync_copy(buf, acc.at[idx], add=True)

    plsc.subcore_barrier()
    pltpu.sync_copy(acc.at[pl.ds(s * NPT, NPT)],
                    out_hbm.at[c].at[pl.ds(s * NPT, NPT)])


def kernel(x, edge_index, batch, W_enc, b_enc, msg1_W, msg1_b, msg2_W, msg2_b,
           upd1_W, upd1_b, upd2_W, upd2_b, W_dec, b_dec):
    src = edge_index[0]
    dst = edge_index[1]
    x8 = jnp.pad(x, ((0, 0), (0, 8 - x.shape[1])))
    we8 = jnp.pad(W_enc, ((0, 8 - W_enc.shape[0]), (0, 0)))
    batc = batch[:, None]
    batr = jnp.broadcast_to(batch[None, :], (8, N))
    zeros = jnp.zeros((NACC, H), _f32)
    wd = jnp.pad(W_dec, ((0, 0), (0, H - W_dec.shape[1])))
    bd = jnp.pad(b_dec, (0, H - b_dec.shape[0]))[None, :]

    h, A, B = _enc(x8, we8, b_enc[None, :],
                   msg1_W[0, :H], msg1_W[0, H:], msg1_b[0][None, :])
    for l in range(NL):
        ag, bg = _gather2(A, B, dst, src)
        m = _edge(ag, bg, msg2_W[l], msg2_b[l][None, :])
        parts = _scatter_add(m, dst, zeros)
        if l < NL - 1:
            h, A, B = _upd(h, parts,
                           upd1_W[l, :H], upd1_W[l, H:], upd1_b[l][None, :],
                           upd2_W[l], upd2_b[l][None, :], batc, batr,
                           msg1_W[l + 1, :H], msg1_W[l + 1, H:],
                           msg1_b[l + 1][None, :])
        else:
            y = _fin(h, parts,
                     upd1_W[l, :H], upd1_W[l, H:], upd1_b[l][None, :],
                     upd2_W[l], upd2_b[l][None, :], batc, batr, wd, bd)
    return y[:, :3]
